# baseline (speedup 1.0000x reference)
import functools

import jax
import jax.numpy as jnp
from jax import lax
from jax.experimental import pallas as pl
from jax.experimental.pallas import tpu as pltpu
from jax.experimental.pallas import tpu_sc as plsc

NUM_LANGUAGES = 4
VOCAB = 100000
DIM = 512
BATCH = 4096
SEQ = 50

N_ROWS = BATCH * SEQ
NC, NS = 2, 16
NW = NC * NS
RPW = N_ROWS // NW
W = 40
NBUF = 4
GPW = RPW // W


def _gather_rows(table_flat, idx3):
    mesh = plsc.VectorSubcoreMesh(core_axis_name="c", subcore_axis_name="s")

    @functools.partial(
        pl.kernel,
        out_type=jax.ShapeDtypeStruct((N_ROWS, DIM), jnp.float32),
        mesh=mesh,
        scratch_types=(
            [pltpu.VMEM((RPW,), jnp.int32)]
            + [pltpu.VMEM((W, DIM), jnp.float32) for _ in range(NBUF)]
            + [pltpu.SemaphoreType.DMA for _ in range(2 * NBUF)]
        ),
    )
    def k(table_hbm, idx_hbm, out_hbm, idx_all, *bufs_and_sems):
        rows = bufs_and_sems[:NBUF]
        gsem = bufs_and_sems[NBUF : 2 * NBUF]
        wsem = bufs_and_sems[2 * NBUF :]
        wid = lax.axis_index("s") * NC + lax.axis_index("c")
        base = pl.multiple_of(wid * RPW, RPW)

        pltpu.sync_copy(idx_hbm.at[wid, 0], idx_all)

        def start_gather(g, b):
            off = pl.multiple_of(g * W, 8)
            pltpu.async_copy(table_hbm.at[idx_all.at[pl.ds(off, W)]], rows[b], gsem[b])

        def start_write(g, b):
            off = pl.multiple_of(base + g * W, 8)
            pltpu.async_copy(rows[b], out_hbm.at[pl.ds(off, W)], wsem[b])

        def drain(sem, b):
            pltpu.make_async_copy(out_hbm.at[pl.ds(0, W)], rows[b], sem).wait()

        for b in range(NBUF):
            start_gather(b, b)

        @pl.loop(0, GPW - NBUF, step=NBUF)
        def _(g0):
            for b in range(NBUF):
                drain(gsem[b], b)
                start_write(g0 + b, b)
            for b in range(NBUF):
                drain(wsem[b], b)
                start_gather(g0 + NBUF + b, b)

        for b in range(NBUF):
            drain(gsem[b], b)
            start_write(GPW - NBUF + b, b)
        for b in range(NBUF):
            drain(wsem[b], b)

    return k(table_flat, idx3)


def kernel(tables, tokens, language_id):
    table_flat = tables.reshape(NUM_LANGUAGES * VOCAB, DIM)
    tok_off = tokens.astype(jnp.int32) + jnp.int32(language_id) * VOCAB
    idx3 = tok_off.T.reshape(NW, 1, RPW)
    rows = _gather_rows(table_flat, idx3)
    shared_embedding = rows.reshape(SEQ, BATCH, DIM).transpose(1, 0, 2)
    language_prediction = jax.random.normal(
        jax.random.key(42), (1, 10), dtype=jnp.float32
    )
    return (shared_embedding, language_prediction)


# speedup vs baseline: 1.0045x; 1.0045x over previous
import functools

import jax
import jax.numpy as jnp
from jax.experimental import pallas as pl
from jax.experimental.pallas import tpu as pltpu
from jax.experimental.pallas import tpu_sc as plsc

NUM_LANGUAGES = 4
VOCAB = 100000
DIM = 512
BATCH = 4096
SEQ = 50


WINDOW = 128
HALF = WINDOW // 2
N_WIN = BATCH * SEQ // WINDOW


def _gather_rows(table_flat, idxp):
    mesh = plsc.VectorSubcoreMesh(core_axis_name="c", subcore_axis_name="s")

    @functools.partial(
        pl.kernel,
        out_type=jax.ShapeDtypeStruct((BATCH * SEQ, DIM), jnp.float32),
        mesh=mesh,
    )
    def k(table_hbm, idx_hbm, out_hbm):
        def body(idx_vmem, out_vmem):
            j = pl.program_id(1)
            idx_s = idx_vmem.at[0, 0, pl.ds(j * HALF, HALF)]
            pltpu.sync_copy(table_hbm.at[idx_s], out_vmem)

        pltpu.emit_pipeline(
            body,
            grid=(N_WIN, 2),
            in_specs=[pl.BlockSpec((1, 1, WINDOW), index_map=lambda i, j: (i, 0, 0))],
            out_specs=[pl.BlockSpec((HALF, DIM), index_map=lambda i, j: (2 * i + j, 0))],
            core_axis_name=("c", "s"),
            dimension_semantics=(pltpu.PARALLEL, pltpu.ARBITRARY),
        )(idx_hbm, out_hbm)

    return k(table_flat, idxp)


def kernel(tables, tokens, language_id):
    table_flat = tables.reshape(NUM_LANGUAGES * VOCAB, DIM)
    tok_off = tokens.astype(jnp.int32) + jnp.int32(language_id) * VOCAB
    idxp = tok_off.T.reshape(N_WIN, 1, WINDOW)
    rows = _gather_rows(table_flat, idxp)
    shared_embedding = rows.reshape(SEQ, BATCH, DIM).transpose(1, 0, 2)
    language_prediction = jax.random.normal(
        jax.random.key(42), (1, 10), dtype=jnp.float32
    )
    return (shared_embedding, language_prediction)
